# baseline (speedup 1.0000x reference)
import functools

import jax
import jax.numpy as jnp
from jax import lax
from jax.experimental import pallas as pl
from jax.experimental.pallas import tpu as pltpu
from jax.experimental.pallas import tpu_sc as plsc

V = 1000000
D = 64
NC = 2
NS = 16
NW = NC * NS
NFULL = V // 128
VTAIL = V - NFULL * 128
PITCH = 128

_MESH = dict(core_axis_name="c", subcore_axis_name="s")


def _wid():
    return lax.axis_index("s") * NC + lax.axis_index("c")


def _iota16():
    return lax.iota(jnp.int32, 16)


def _fmt_body(tab_t, tail_pad, scratch, in_v, tr_v, *sems):
    isems, osems = sems[:2], sems[2:4]
    wid = _wid()
    n_w = (NFULL - 1 - wid) // NW + 1

    def col(i):
        return (wid + i * NW) * 128

    def in_start(i, b):
        pltpu.make_async_copy(
            tab_t.at[:, pl.ds(col(i), 128)], in_v.at[b], isems[b]).start()

    def in_wait(i, b):
        pltpu.make_async_copy(
            tab_t.at[:, pl.ds(col(i), 128)], in_v.at[b], isems[b]).wait()

    def out_start(i, b):
        pltpu.make_async_copy(
            tr_v.at[b], scratch.at[pl.ds(col(i), 128)], osems[b]).start()

    def out_wait(i, b):
        pltpu.make_async_copy(
            tr_v.at[b], scratch.at[pl.ds(col(i), 128)], osems[b]).wait()

    in_start(0, 0)
    in_start(1, 1)
    rows = [_iota16() + 16 * k for k in range(4)]

    def transpose_block(b, width):
        def tbody(v, carry):
            cols = jnp.full((16,), v, jnp.int32)
            for k in range(4):
                tr_v[b, v, pl.ds(16 * k, 16)] = plsc.load_gather(
                    in_v.at[b], [rows[k], cols])
            return carry
        lax.fori_loop(0, width, tbody, 0)

    def body(i, carry):
        for b in range(2):
            ib = i * 2 + b
            in_wait(ib, b)

            @pl.when(ib >= 2)
            def _():
                out_wait(ib - 2, b)

            transpose_block(b, 128)
            out_start(ib, b)

            @pl.when(ib + 2 < n_w)
            def _():
                in_start(ib + 2, b)
        return carry

    lax.fori_loop(0, n_w // 2, body, 0)

    @pl.when(n_w % 2 == 1)
    def _():
        ib = n_w - 1
        in_wait(ib, 0)
        out_wait(ib - 2, 0)
        transpose_block(0, 128)
        out_start(ib, 0)

    @pl.when(n_w % 2 == 1)
    def _():
        out_wait(n_w - 1, 0)
        out_wait(n_w - 2, 1)

    @pl.when(n_w % 2 == 0)
    def _():
        out_wait(n_w - 2, 0)
        out_wait(n_w - 1, 1)

    @pl.when(wid == NFULL % NW)
    def _():
        pltpu.sync_copy(tail_pad, tr_v.at[0, pl.ds(0, VTAIL)])
        pltpu.sync_copy(tr_v.at[0, pl.ds(0, VTAIL)],
                        scratch.at[pl.ds(NFULL * 128, VTAIL)])


def _lookup_body(ids_t, scratch, out_t, idx_v, g_v, ot_v, *sems):
    hist = ids_t.shape[0]
    bpw = ids_t.shape[1] // NW
    gsems, ssems = sems[:2], sems[2:4]
    wid = _wid()
    base = wid * bpw
    pltpu.sync_copy(ids_t.at[:, pl.ds(base, bpw)], idx_v)

    def g_start(h, b):
        pltpu.make_async_copy(
            scratch.at[idx_v.at[h]], g_v.at[b], gsems[b]).start()

    def g_wait(h, b):
        pltpu.make_async_copy(
            scratch.at[idx_v.at[h]], g_v.at[b], gsems[b]).wait()

    def s_start(h, b):
        pltpu.make_async_copy(
            ot_v.at[b], out_t.at[h, :, pl.ds(base, bpw)], ssems[b]).start()

    def s_wait(h, b):
        pltpu.make_async_copy(
            ot_v.at[b], out_t.at[h, :, pl.ds(base, bpw)], ssems[b]).wait()

    g_start(0, 0)
    g_start(1, 1)
    rows = [_iota16() + 16 * k for k in range(8)]

    def body(i, carry):
        for b in range(2):
            h = i * 2 + b
            g_wait(h, b)

            @pl.when(h >= 2)
            def _():
                s_wait(h - 2, b)

            def tbody(f, c2):
                cols = jnp.full((16,), f, jnp.int32)
                for k in range(8):
                    ot_v[b, f, pl.ds(16 * k, 16)] = plsc.load_gather(
                        g_v.at[b], [rows[k], cols])
                return c2
            lax.fori_loop(0, D, tbody, 0)
            s_start(h, b)

            @pl.when(h + 2 < hist)
            def _():
                g_start(h + 2, b)
        return carry

    lax.fori_loop(0, hist // 2, body, 0)
    s_wait(hist - 2, 0)
    s_wait(hist - 1, 1)


@jax.jit
def kernel(token_ids, embeddings):
    bsz, hist = token_ids.shape
    mesh = plsc.VectorSubcoreMesh(num_cores=NC, num_subcores=NS, **_MESH)
    fmt = pl.kernel(
        _fmt_body,
        out_type=jax.ShapeDtypeStruct((V, PITCH), jnp.float32),
        mesh=mesh,
        scratch_types=[
            pltpu.VMEM((2, D, 128), jnp.float32),
            pltpu.VMEM((2, 128, PITCH), jnp.float32),
        ] + [pltpu.SemaphoreType.DMA] * 4,
        compiler_params=pltpu.CompilerParams(use_tc_tiling_on_sc=True, needs_layout_passes=False),
    )
    tail_pad = jnp.pad(embeddings[NFULL * 128:, :], ((0, 0), (0, PITCH - D)))
    scratch = fmt(jnp.transpose(embeddings), tail_pad)
    lookup = pl.kernel(
        _lookup_body,
        out_type=jax.ShapeDtypeStruct((hist, D, bsz), jnp.float32),
        mesh=mesh,
        scratch_types=[
            pltpu.VMEM((hist, bsz // NW), jnp.int32),
            pltpu.VMEM((2, bsz // NW, PITCH), jnp.float32),
            pltpu.VMEM((2, D, bsz // NW), jnp.float32),
        ] + [pltpu.SemaphoreType.DMA] * 4,
        compiler_params=pltpu.CompilerParams(use_tc_tiling_on_sc=True, needs_layout_passes=False),
    )
    out_t = lookup(jnp.transpose(token_ids).astype(jnp.int32), scratch)
    return jnp.transpose(out_t, (2, 0, 1))


# speedup vs baseline: 1.8858x; 1.8858x over previous
import functools

import jax
import jax.numpy as jnp
from jax import lax
from jax.experimental import pallas as pl
from jax.experimental.pallas import tpu as pltpu
from jax.experimental.pallas import tpu_sc as plsc

V = 1000000
D = 64
NC = 2
NS = 16
NW = NC * NS
NFULL = V // 128
VTAIL = V - NFULL * 128
PITCH = 128

_MESH = dict(core_axis_name="c", subcore_axis_name="s")


def _wid():
    return lax.axis_index("s") * NC + lax.axis_index("c")


def _iota16():
    return lax.iota(jnp.int32, 16)


def _fmt_body(tab_t, tail_pad, scratch, in_v, tr_v, *sems):
    isems, osems = sems[:2], sems[2:4]
    wid = _wid()
    n_w = (NFULL - 1 - wid) // NW + 1

    def col(i):
        return (wid + i * NW) * 128

    def in_start(i, b):
        pltpu.make_async_copy(
            tab_t.at[:, pl.ds(col(i), 128)], in_v.at[b], isems[b]).start()

    def in_wait(i, b):
        pltpu.make_async_copy(
            tab_t.at[:, pl.ds(col(i), 128)], in_v.at[b], isems[b]).wait()

    def out_start(i, b):
        pltpu.make_async_copy(
            tr_v.at[b], scratch.at[pl.ds(col(i), 128)], osems[b]).start()

    def out_wait(i, b):
        pltpu.make_async_copy(
            tr_v.at[b], scratch.at[pl.ds(col(i), 128)], osems[b]).wait()

    in_start(0, 0)
    in_start(1, 1)
    rows = [_iota16() + 16 * k for k in range(4)]

    def transpose_block(b, width):
        @plsc.parallel_loop(0, width, unroll=8)
        def _(v):
            cols = jnp.full((16,), v, jnp.int32)
            for k in range(4):
                tr_v[b, v, pl.ds(16 * k, 16)] = plsc.load_gather(
                    in_v.at[b], [rows[k], cols])

    def body(i, carry):
        for b in range(2):
            ib = i * 2 + b
            in_wait(ib, b)

            @pl.when(ib >= 2)
            def _():
                out_wait(ib - 2, b)

            transpose_block(b, 128)
            out_start(ib, b)

            @pl.when(ib + 2 < n_w)
            def _():
                in_start(ib + 2, b)
        return carry

    lax.fori_loop(0, n_w // 2, body, 0)

    @pl.when(n_w % 2 == 1)
    def _():
        ib = n_w - 1
        in_wait(ib, 0)
        out_wait(ib - 2, 0)
        transpose_block(0, 128)
        out_start(ib, 0)

    @pl.when(n_w % 2 == 1)
    def _():
        out_wait(n_w - 1, 0)
        out_wait(n_w - 2, 1)

    @pl.when(n_w % 2 == 0)
    def _():
        out_wait(n_w - 2, 0)
        out_wait(n_w - 1, 1)

    @pl.when(wid == NFULL % NW)
    def _():
        pltpu.sync_copy(tail_pad, tr_v.at[0, pl.ds(0, VTAIL)])
        pltpu.sync_copy(tr_v.at[0, pl.ds(0, VTAIL)],
                        scratch.at[pl.ds(NFULL * 128, VTAIL)])


def _lookup_body(ids_t, scratch, out_t, idx_v, g_v, ot_v, *sems):
    hist = ids_t.shape[0]
    bpw = ids_t.shape[1] // NW
    gsems, ssems = sems[:2], sems[2:4]
    wid = _wid()
    base = wid * bpw
    pltpu.sync_copy(ids_t.at[:, pl.ds(base, bpw)], idx_v)

    def g_start(h, b):
        pltpu.make_async_copy(
            scratch.at[idx_v.at[h]], g_v.at[b], gsems[b]).start()

    def g_wait(h, b):
        pltpu.make_async_copy(
            scratch.at[idx_v.at[h]], g_v.at[b], gsems[b]).wait()

    def s_start(h, b):
        pltpu.make_async_copy(
            ot_v.at[b], out_t.at[h, :, pl.ds(base, bpw)], ssems[b]).start()

    def s_wait(h, b):
        pltpu.make_async_copy(
            ot_v.at[b], out_t.at[h, :, pl.ds(base, bpw)], ssems[b]).wait()

    g_start(0, 0)
    g_start(1, 1)
    rows = [_iota16() + 16 * k for k in range(8)]

    def body(i, carry):
        for b in range(2):
            h = i * 2 + b
            g_wait(h, b)

            @pl.when(h >= 2)
            def _():
                s_wait(h - 2, b)

            @plsc.parallel_loop(0, D, unroll=4)
            def _(f):
                cols = jnp.full((16,), f, jnp.int32)
                for k in range(8):
                    ot_v[b, f, pl.ds(16 * k, 16)] = plsc.load_gather(
                        g_v.at[b], [rows[k], cols])
            s_start(h, b)

            @pl.when(h + 2 < hist)
            def _():
                g_start(h + 2, b)
        return carry

    lax.fori_loop(0, hist // 2, body, 0)
    s_wait(hist - 2, 0)
    s_wait(hist - 1, 1)


@jax.jit
def kernel(token_ids, embeddings):
    bsz, hist = token_ids.shape
    mesh = plsc.VectorSubcoreMesh(num_cores=NC, num_subcores=NS, **_MESH)
    fmt = pl.kernel(
        _fmt_body,
        out_type=jax.ShapeDtypeStruct((V, PITCH), jnp.float32),
        mesh=mesh,
        scratch_types=[
            pltpu.VMEM((2, D, 128), jnp.float32),
            pltpu.VMEM((2, 128, PITCH), jnp.float32),
        ] + [pltpu.SemaphoreType.DMA] * 4,
        compiler_params=pltpu.CompilerParams(use_tc_tiling_on_sc=True, needs_layout_passes=False),
    )
    tail_pad = jnp.pad(embeddings[NFULL * 128:, :], ((0, 0), (0, PITCH - D)))
    scratch = fmt(jnp.transpose(embeddings), tail_pad)
    lookup = pl.kernel(
        _lookup_body,
        out_type=jax.ShapeDtypeStruct((hist, D, bsz), jnp.float32),
        mesh=mesh,
        scratch_types=[
            pltpu.VMEM((hist, bsz // NW), jnp.int32),
            pltpu.VMEM((2, bsz // NW, PITCH), jnp.float32),
            pltpu.VMEM((2, D, bsz // NW), jnp.float32),
        ] + [pltpu.SemaphoreType.DMA] * 4,
        compiler_params=pltpu.CompilerParams(use_tc_tiling_on_sc=True, needs_layout_passes=False),
    )
    out_t = lookup(jnp.transpose(token_ids).astype(jnp.int32), scratch)
    return jnp.transpose(out_t, (2, 0, 1))


# speedup vs baseline: 2.2946x; 1.2168x over previous
import functools

import jax
import jax.numpy as jnp
from jax import lax
from jax.experimental import pallas as pl
from jax.experimental.pallas import tpu as pltpu
from jax.experimental.pallas import tpu_sc as plsc

V = 1000000
D = 64
NC = 2
NS = 16
NW = NC * NS
PITCH = 128
NG = 3
NO = 2

_MESH = dict(core_axis_name="c", subcore_axis_name="s")


def _lookup_body(ids_t, scratch, out_t, idx_v, g_v, ot_v, *sems):
    hist = ids_t.shape[0]
    bpw = ids_t.shape[1] // NW
    gsems, ssems = sems[:NG], sems[NG:NG + NO]
    wid = lax.axis_index("s") * NC + lax.axis_index("c")
    base = wid * bpw
    pltpu.sync_copy(ids_t.at[:, pl.ds(base, bpw)], idx_v)

    def g_start(h, b):
        pltpu.make_async_copy(
            scratch.at[idx_v.at[h]], g_v.at[b], gsems[b]).start()

    def g_wait(h, b):
        pltpu.make_async_copy(
            scratch.at[idx_v.at[h]], g_v.at[b], gsems[b]).wait()

    def s_start(h, b):
        pltpu.make_async_copy(
            ot_v.at[b], out_t.at[h, :, pl.ds(base, bpw)], ssems[b]).start()

    def s_wait(h, b):
        pltpu.make_async_copy(
            ot_v.at[b], out_t.at[h, :, pl.ds(base, bpw)], ssems[b]).wait()

    for b in range(NG):
        g_start(b, b)
    rows = [lax.iota(jnp.int32, 16) + 16 * k for k in range(8)]

    def step(h, bg, bo, first):
        g_wait(h, bg)
        if not first:
            s_wait(h - NO, bo)

        @plsc.parallel_loop(0, D, unroll=4)
        def _(f):
            cols = jnp.full((16,), f, jnp.int32)
            for k in range(8):
                ot_v[bo, f, pl.ds(16 * k, 16)] = plsc.load_gather(
                    g_v.at[bg], [rows[k], cols])
        s_start(h, bo)

        @pl.when(h + NG < hist)
        def _():
            g_start(h + NG, bg)

    period = NG * NO
    nmain = (hist - 2) // period

    def body(i, carry):
        for j in range(period):
            h = i * period + j
            step(h, j % NG, j % NO, False)
        return carry

    for j in range(period):
        step(j, j % NG, j % NO, j < NO)

    def body2(i, carry):
        for j in range(period):
            h = (i + 1) * period + j
            step(h, j % NG, j % NO, False)
        return carry

    lax.fori_loop(0, nmain - 1, body2, 0)
    for h in range(nmain * period, hist):
        step(h, h % NG, h % NO, False)
    s_wait(hist - 2, (hist - 2) % NO)
    s_wait(hist - 1, (hist - 1) % NO)


@jax.jit
def kernel(token_ids, embeddings):
    bsz, hist = token_ids.shape
    mesh = plsc.VectorSubcoreMesh(num_cores=NC, num_subcores=NS, **_MESH)
    scratch = jnp.pad(embeddings, ((0, 0), (0, PITCH - D)))
    lookup = pl.kernel(
        _lookup_body,
        out_type=jax.ShapeDtypeStruct((hist, D, bsz), jnp.float32),
        mesh=mesh,
        scratch_types=[
            pltpu.VMEM((hist, bsz // NW), jnp.int32),
            pltpu.VMEM((NG, bsz // NW, PITCH), jnp.float32),
            pltpu.VMEM((NO, D, bsz // NW), jnp.float32),
        ] + [pltpu.SemaphoreType.DMA] * (NG + NO),
        compiler_params=pltpu.CompilerParams(
            use_tc_tiling_on_sc=True, needs_layout_passes=False),
    )
    out_t = lookup(jnp.transpose(token_ids).astype(jnp.int32), scratch)
    return jnp.transpose(out_t, (2, 0, 1))
